# Initial kernel scaffold; baseline (speedup 1.0000x reference)
import jax
import jax.numpy as jnp
from jax.experimental import pallas as pl


def kernel(h, adj_mat, W, attn_a):
    raise NotImplementedError("write your pallas kernel here")


# speedup vs baseline: 1.4449x; 1.4449x over previous
import functools

import jax
import jax.numpy as jnp
from jax.experimental import pallas as pl
from jax.experimental.pallas import tpu as pltpu

SLOPE = 0.2


def _prep_kernel(h_ref, w_ref, a_ref, g_ref, u_ref, u2_ref, v_ref, v2_ref,
                 *, n_heads, n_hidden):
    g = jnp.dot(h_ref[...], w_ref[...], preferred_element_type=jnp.float32)
    g_ref[...] = g
    a = a_ref[...]
    for hd in range(n_heads):
        gh = g[:, hd * n_hidden:(hd + 1) * n_hidden]
        e_l = jax.lax.dot_general(gh, a[0:1, :], (((1,), (1,)), ((), ())),
                                  preferred_element_type=jnp.float32)
        e_r = jax.lax.dot_general(a[1:2, :], gh, (((1,), (1,)), ((), ())),
                                  preferred_element_type=jnp.float32)
        mmax = jnp.max(e_r)
        t = e_l + mmax
        m = jnp.maximum(t, SLOPE * t)
        u_ref[:, hd:hd + 1] = jnp.exp(e_l - m)
        u2_ref[:, hd:hd + 1] = jnp.exp(SLOPE * e_l - m)
        v_ref[hd:hd + 1, :] = jnp.exp(e_r)
        v2_ref[hd:hd + 1, :] = jnp.exp(SLOPE * e_r)


def _attn_kernel(adj_ref, g_ref, u_ref, u2_ref, v_ref, v2_ref, o_ref, den_ref,
                 *, n_heads, n_hidden):
    j = pl.program_id(1)
    nj = pl.num_programs(1)

    @pl.when(j == 0)
    def _init():
        o_ref[...] = jnp.zeros_like(o_ref)
        den_ref[...] = jnp.zeros_like(den_ref)

    adjf = adj_ref[...].astype(jnp.float32)
    for hd in range(n_heads):
        u = u_ref[:, hd:hd + 1]
        u2 = u2_ref[:, hd:hd + 1]
        v = v_ref[hd:hd + 1, :]
        v2 = v2_ref[hd:hd + 1, :]
        p = jnp.maximum(u * v, u2 * v2) * adjf
        gh = g_ref[:, hd * n_hidden:(hd + 1) * n_hidden]
        sl = slice(hd * n_hidden, (hd + 1) * n_hidden)
        o_ref[:, sl] += jnp.dot(p, gh, preferred_element_type=jnp.float32)
        den_ref[:, hd:hd + 1] += jnp.sum(p, axis=1, keepdims=True)

    @pl.when(j == nj - 1)
    def _final():
        for hd in range(n_heads):
            sl = slice(hd * n_hidden, (hd + 1) * n_hidden)
            o_ref[:, sl] = o_ref[:, sl] / den_ref[:, hd:hd + 1]


def kernel(h, adj_mat, W, attn_a):
    n, in_f = h.shape
    out_f = W.shape[1]
    n_hidden = attn_a.shape[0] // 2
    n_heads = out_f // n_hidden

    adj2 = adj_mat.reshape(n, n)
    a2 = attn_a.reshape(2, n_hidden)

    f32 = jnp.float32
    g, u, u2, v, v2 = pl.pallas_call(
        functools.partial(_prep_kernel, n_heads=n_heads, n_hidden=n_hidden),
        out_shape=(
            jax.ShapeDtypeStruct((n, out_f), f32),
            jax.ShapeDtypeStruct((n, n_heads), f32),
            jax.ShapeDtypeStruct((n, n_heads), f32),
            jax.ShapeDtypeStruct((n_heads, n), f32),
            jax.ShapeDtypeStruct((n_heads, n), f32),
        ),
    )(h, W, a2)

    bi = min(512, n)
    bj = min(512, n)
    ni, nj = n // bi, n // bj

    out = pl.pallas_call(
        functools.partial(_attn_kernel, n_heads=n_heads, n_hidden=n_hidden),
        grid=(ni, nj),
        in_specs=[
            pl.BlockSpec((bi, bj), lambda i, j: (i, j)),
            pl.BlockSpec((bj, out_f), lambda i, j: (j, 0)),
            pl.BlockSpec((bi, n_heads), lambda i, j: (i, 0)),
            pl.BlockSpec((bi, n_heads), lambda i, j: (i, 0)),
            pl.BlockSpec((n_heads, bj), lambda i, j: (0, j)),
            pl.BlockSpec((n_heads, bj), lambda i, j: (0, j)),
        ],
        out_specs=pl.BlockSpec((bi, out_f), lambda i, j: (i, 0)),
        out_shape=jax.ShapeDtypeStruct((n, out_f), f32),
        scratch_shapes=[pltpu.VMEM((bi, n_heads), f32)],
        compiler_params=pltpu.CompilerParams(
            dimension_semantics=("arbitrary", "arbitrary"),
        ),
    )(adj2, g, u, u2, v, v2)
    return out


# speedup vs baseline: 1.6055x; 1.1111x over previous
import functools

import jax
import jax.numpy as jnp
from jax.experimental import pallas as pl
from jax.experimental.pallas import tpu as pltpu

SLOPE = 0.2
HEAD_W = 64


def _prep_kernel(h_ref, w_ref, a_ref, gx_ref, u_ref, u2_ref, v_ref, v2_ref,
                 *, n_heads, n_hidden):
    n = h_ref.shape[0]
    g = jnp.dot(h_ref[...], w_ref[...], preferred_element_type=jnp.float32)
    a = a_ref[...]
    ones = jnp.ones((n, 1), jnp.float32)
    zeros = jnp.zeros((n, HEAD_W - n_hidden - 1), jnp.float32)
    parts = []
    for hd in range(n_heads):
        gh = g[:, hd * n_hidden:(hd + 1) * n_hidden]
        parts += [gh, ones, zeros]
        e_l = jax.lax.dot_general(gh, a[0:1, :], (((1,), (1,)), ((), ())),
                                  preferred_element_type=jnp.float32)
        e_r = jax.lax.dot_general(a[1:2, :], gh, (((1,), (1,)), ((), ())),
                                  preferred_element_type=jnp.float32)
        mmax = jnp.max(e_r)
        t = e_l + mmax
        m = jnp.maximum(t, SLOPE * t)
        u_ref[:, hd:hd + 1] = jnp.exp(e_l - m)
        u2_ref[:, hd:hd + 1] = jnp.exp(SLOPE * e_l - m)
        v_ref[hd:hd + 1, :] = jnp.exp(e_r)
        v2_ref[hd:hd + 1, :] = jnp.exp(SLOPE * e_r)
    gx_ref[...] = jnp.concatenate(parts, axis=1).astype(jnp.bfloat16)


def _attn_kernel(adj_ref, gx_ref, u_ref, u2_ref, v_ref, v2_ref, o_ref, acc_ref,
                 *, n_heads, n_hidden):
    j = pl.program_id(1)
    nj = pl.num_programs(1)

    @pl.when(j == 0)
    def _init():
        acc_ref[...] = jnp.zeros_like(acc_ref)

    adj = adj_ref[...]
    for hd in range(n_heads):
        u = u_ref[:, hd:hd + 1]
        u2 = u2_ref[:, hd:hd + 1]
        v = v_ref[hd:hd + 1, :]
        v2 = v2_ref[hd:hd + 1, :]
        p = jnp.where(adj, jnp.maximum(u * v, u2 * v2), 0.0).astype(jnp.bfloat16)
        gx = gx_ref[:, hd * HEAD_W:(hd + 1) * HEAD_W]
        acc_ref[:, hd * HEAD_W:(hd + 1) * HEAD_W] += jnp.dot(
            p, gx, preferred_element_type=jnp.float32)

    @pl.when(j == nj - 1)
    def _final():
        for hd in range(n_heads):
            num = acc_ref[:, hd * HEAD_W:hd * HEAD_W + n_hidden]
            den = acc_ref[:, hd * HEAD_W + n_hidden:hd * HEAD_W + n_hidden + 1]
            o_ref[:, hd * n_hidden:(hd + 1) * n_hidden] = num * (1.0 / den)


def kernel(h, adj_mat, W, attn_a):
    n, in_f = h.shape
    out_f = W.shape[1]
    n_hidden = attn_a.shape[0] // 2
    n_heads = out_f // n_hidden

    adj2 = adj_mat.reshape(n, n)
    a2 = attn_a.reshape(2, n_hidden)

    f32 = jnp.float32
    gx, u, u2, v, v2 = pl.pallas_call(
        functools.partial(_prep_kernel, n_heads=n_heads, n_hidden=n_hidden),
        out_shape=(
            jax.ShapeDtypeStruct((n, n_heads * HEAD_W), jnp.bfloat16),
            jax.ShapeDtypeStruct((n, n_heads), f32),
            jax.ShapeDtypeStruct((n, n_heads), f32),
            jax.ShapeDtypeStruct((n_heads, n), f32),
            jax.ShapeDtypeStruct((n_heads, n), f32),
        ),
    )(h, W, a2)

    bi = min(512, n)
    bj = min(512, n)
    ni, nj = n // bi, n // bj

    out = pl.pallas_call(
        functools.partial(_attn_kernel, n_heads=n_heads, n_hidden=n_hidden),
        grid=(ni, nj),
        in_specs=[
            pl.BlockSpec((bi, bj), lambda i, j: (i, j)),
            pl.BlockSpec((bj, n_heads * HEAD_W), lambda i, j: (j, 0)),
            pl.BlockSpec((bi, n_heads), lambda i, j: (i, 0)),
            pl.BlockSpec((bi, n_heads), lambda i, j: (i, 0)),
            pl.BlockSpec((n_heads, bj), lambda i, j: (0, j)),
            pl.BlockSpec((n_heads, bj), lambda i, j: (0, j)),
        ],
        out_specs=pl.BlockSpec((bi, out_f), lambda i, j: (i, 0)),
        out_shape=jax.ShapeDtypeStruct((n, out_f), f32),
        scratch_shapes=[pltpu.VMEM((bi, n_heads * HEAD_W), f32)],
        compiler_params=pltpu.CompilerParams(
            dimension_semantics=("arbitrary", "arbitrary"),
        ),
    )(adj2, gx, u, u2, v, v2)
    return out


# speedup vs baseline: 2.5603x; 1.5948x over previous
import functools

import jax
import jax.numpy as jnp
from jax.experimental import pallas as pl
from jax.experimental.pallas import tpu as pltpu

SLOPE = 0.2
HEAD_W = 64


def _fused_kernel(h_ref, w_ref, a_ref, adj_ref, o_ref,
                  gx_s, u_s, u2_s, v_s, v2_s, acc_s,
                  *, n, bi, bj, n_heads, n_hidden):
    i = pl.program_id(0)
    j = pl.program_id(1)
    nj = pl.num_programs(1)

    @pl.when((i == 0) & (j == 0))
    def _prep():
        g = jnp.dot(h_ref[...], w_ref[...], preferred_element_type=jnp.float32)
        a = a_ref[...]
        ones = jnp.ones((n, 1), jnp.float32)
        zeros = jnp.zeros((n, HEAD_W - n_hidden - 1), jnp.float32)
        parts = []
        for hd in range(n_heads):
            gh = g[:, hd * n_hidden:(hd + 1) * n_hidden]
            parts += [gh, ones, zeros]
            e_l = jax.lax.dot_general(gh, a[0:1, :], (((1,), (1,)), ((), ())),
                                      preferred_element_type=jnp.float32)
            e_r = jax.lax.dot_general(a[1:2, :], gh, (((1,), (1,)), ((), ())),
                                      preferred_element_type=jnp.float32)
            mmax = jnp.max(e_r)
            t = e_l + mmax
            m = jnp.maximum(t, SLOPE * t)
            u_s[:, hd:hd + 1] = jnp.exp(e_l - m)
            u2_s[:, hd:hd + 1] = jnp.exp(SLOPE * e_l - m)
            v_s[hd:hd + 1, :] = jnp.exp(e_r)
            v2_s[hd:hd + 1, :] = jnp.exp(SLOPE * e_r)
        gx_s[...] = jnp.concatenate(parts, axis=1).astype(jnp.bfloat16)

    @pl.when(j == 0)
    def _init():
        acc_s[...] = jnp.zeros_like(acc_s)

    adj = adj_ref[...] != 0
    for hd in range(n_heads):
        uh = u_s[pl.ds(i * bi, bi), hd:hd + 1]
        u2h = u2_s[pl.ds(i * bi, bi), hd:hd + 1]
        vh = v_s[hd:hd + 1, pl.ds(j * bj, bj)]
        v2h = v2_s[hd:hd + 1, pl.ds(j * bj, bj)]
        p = jnp.where(adj, jnp.maximum(uh * vh, u2h * v2h),
                      0.0).astype(jnp.bfloat16)
        gxh = gx_s[pl.ds(j * bj, bj), hd * HEAD_W:(hd + 1) * HEAD_W]
        acc_s[:, hd * HEAD_W:(hd + 1) * HEAD_W] += jnp.dot(
            p, gxh, preferred_element_type=jnp.float32)

    @pl.when(j == nj - 1)
    def _final():
        for hd in range(n_heads):
            num = acc_s[:, hd * HEAD_W:hd * HEAD_W + n_hidden]
            den = acc_s[:, hd * HEAD_W + n_hidden:hd * HEAD_W + n_hidden + 1]
            o_ref[:, hd * n_hidden:(hd + 1) * n_hidden] = num * (1.0 / den)


def kernel(h, adj_mat, W, attn_a):
    n, in_f = h.shape
    out_f = W.shape[1]
    n_hidden = attn_a.shape[0] // 2
    n_heads = out_f // n_hidden

    a2 = attn_a.reshape(2, n_hidden)
    adj8 = adj_mat.astype(jnp.int8).reshape(n, n)

    bi = min(512, n)
    bj = min(1024, n)
    ni, nj = n // bi, n // bj

    f32 = jnp.float32
    bf16 = jnp.bfloat16
    out = pl.pallas_call(
        functools.partial(_fused_kernel, n=n, bi=bi, bj=bj,
                          n_heads=n_heads, n_hidden=n_hidden),
        grid=(ni, nj),
        in_specs=[
            pl.BlockSpec((n, in_f), lambda i, j: (0, 0)),
            pl.BlockSpec((in_f, out_f), lambda i, j: (0, 0)),
            pl.BlockSpec((2, n_hidden), lambda i, j: (0, 0)),
            pl.BlockSpec((bi, bj), lambda i, j: (i, j)),
        ],
        out_specs=pl.BlockSpec((bi, out_f), lambda i, j: (i, 0)),
        out_shape=jax.ShapeDtypeStruct((n, out_f), f32),
        scratch_shapes=[
            pltpu.VMEM((n, n_heads * HEAD_W), bf16),
            pltpu.VMEM((n, n_heads), f32),
            pltpu.VMEM((n, n_heads), f32),
            pltpu.VMEM((n_heads, n), f32),
            pltpu.VMEM((n_heads, n), f32),
            pltpu.VMEM((bi, n_heads * HEAD_W), f32),
        ],
        compiler_params=pltpu.CompilerParams(
            dimension_semantics=("arbitrary", "arbitrary"),
        ),
    )(h, W, a2, adj8)
    return out
